# Initial kernel scaffold; baseline (speedup 1.0000x reference)
import functools

import jax
import jax.numpy as jnp
from jax import lax
from jax.experimental import pallas as pl
from jax.experimental.pallas import tpu as pltpu
from jax.experimental.pallas import tpu_sc as plsc

F32 = jnp.float32
I32 = jnp.int32


def _sc_gather(table, idx, k):
    n = idx.shape[0]
    d = table.shape[1]
    info = plsc.get_sparse_core_info()
    nc, ns = info.num_cores, info.num_subcores
    nw = nc * ns
    per_w = n // nw
    nblk = per_w // k

    mesh = plsc.VectorSubcoreMesh(core_axis_name="c", subcore_axis_name="s")

    @functools.partial(
        pl.kernel,
        out_type=jax.ShapeDtypeStruct((n, d), F32),
        mesh=mesh,
        scratch_types=[
            pltpu.VMEM((k,), I32),
            pltpu.VMEM((k, d), F32),
            pltpu.SemaphoreType.DMA,
        ],
    )
    def gk(table_hbm, idx_hbm, out_hbm, idx_v, rows_v, sem):
        wid = lax.axis_index("s") * nc + lax.axis_index("c")
        base = wid * per_w

        def body(j, carry):
            off = base + j * k
            pltpu.sync_copy(idx_hbm.at[pl.ds(off, k)], idx_v)
            pltpu.async_copy(table_hbm.at[idx_v], rows_v, sem).wait()
            pltpu.sync_copy(rows_v, out_hbm.at[pl.ds(off, k)])
            return carry

        lax.fori_loop(0, nblk, body, 0)

    return gk(table, idx)


def _sc_scatter_add(ext, dst_idx, nd, k):
    e, dext = ext.shape
    info = plsc.get_sparse_core_info()
    nc, ns = info.num_cores, info.num_subcores
    per_core = e // nc
    per_tile = per_core // ns
    nblk = per_tile // k
    rows_per_tile = nd // ns

    zeros = jnp.zeros((rows_per_tile, dext), F32)
    mesh = plsc.VectorSubcoreMesh(core_axis_name="c", subcore_axis_name="s")

    @functools.partial(
        pl.kernel,
        out_type=(
            jax.ShapeDtypeStruct((nd, dext), F32),
            jax.ShapeDtypeStruct((nd, dext), F32),
        ),
        mesh=mesh,
        scratch_types=[
            pltpu.VMEM((k,), I32),
            pltpu.VMEM((k, dext), F32),
            pltpu.SemaphoreType.DMA,
            pltpu.VMEM_SHARED((nd, dext), F32),
        ],
    )
    def sk(ext_hbm, dst_hbm, z_hbm, out0, out1, idx_v, rows_v, sem, acc):
        cid = lax.axis_index("c")
        sid = lax.axis_index("s")
        my_rows = pl.ds(sid * rows_per_tile, rows_per_tile)
        pltpu.sync_copy(z_hbm, acc.at[my_rows])
        plsc.subcore_barrier()

        base = cid * per_core + sid * per_tile

        def body(j, carry):
            off = base + j * k
            pltpu.sync_copy(dst_hbm.at[pl.ds(off, k)], idx_v)
            pltpu.sync_copy(ext_hbm.at[pl.ds(off, k)], rows_v)
            pltpu.sync_copy(rows_v, acc.at[idx_v], add=True)
            return carry

        lax.fori_loop(0, nblk, body, 0)
        plsc.subcore_barrier()

        @pl.when(cid == 0)
        def _():
            pltpu.sync_copy(acc.at[my_rows], out0.at[my_rows])

        @pl.when(cid == 1)
        def _():
            pltpu.sync_copy(acc.at[my_rows], out1.at[my_rows])

    return sk(ext, dst_idx, zeros)


def _tc_qn(src_data, wqn_t, bqp, nd, b):
    def body(q_ref, w_ref, b_ref, o_ref):
        o_ref[...] = (
            jnp.dot(q_ref[...], w_ref[...], preferred_element_type=F32) + b_ref[...]
        )

    return pl.pallas_call(
        body,
        grid=(nd // b,),
        in_specs=[
            pl.BlockSpec((b, 128), lambda i: (i, 0)),
            pl.BlockSpec((128, 128), lambda i: (0, 0)),
            pl.BlockSpec((1, 128), lambda i: (0, 0)),
        ],
        out_specs=pl.BlockSpec((b, 128), lambda i: (i, 0)),
        out_shape=jax.ShapeDtypeStruct((nd, 128), F32),
        compiler_params=pltpu.CompilerParams(
            dimension_semantics=("parallel",)
        ),
    )(src_data, wqn_t, bqp)


def _tc_edge_pass(src_data, qe, edge_feat, dt2, wk_parts, wv_parts, twp, tbp,
                  e, nd, b, dh, dext):
    wkn_t, wke_t, wkt_t, bk2 = wk_parts
    wvn_t, wve_t, wvt_t, bv2 = wv_parts
    row_off = nd // b
    de = edge_feat.shape[1]

    def body(kv_ref, qe_ref, ef_ref, dt_ref,
             wkn, wke, wkt, bkr, wvn, wve, wvt, bvr, twr, tbr, o_ref):
        tf = jnp.cos(dt_ref[...] * twr[...] + tbr[...])
        kv = kv_ref[...]
        ef = ef_ref[...]
        k_mat = (
            jnp.dot(kv, wkn[...], preferred_element_type=F32)
            + jnp.dot(ef, wke[...], preferred_element_type=F32)
            + jnp.dot(tf, wkt[...], preferred_element_type=F32)
            + bkr[...]
        )
        v_mat = (
            jnp.dot(kv, wvn[...], preferred_element_type=F32)
            + jnp.dot(ef, wve[...], preferred_element_type=F32)
            + jnp.dot(tf, wvt[...], preferred_element_type=F32)
            + bvr[...]
        )
        s = qe_ref[...] * k_mat
        d0 = jnp.sum(s[:, :dh], axis=1, keepdims=True)
        d1 = jnp.sum(s[:, dh:], axis=1, keepdims=True)
        l0 = jnp.where(d0 >= 0, d0, 0.2 * d0)
        l1 = jnp.where(d1 >= 0, d1, 0.2 * d1)
        ex0 = jnp.exp(jnp.minimum(l0, 75.0))
        ex1 = jnp.exp(jnp.minimum(l1, 75.0))
        scale = jnp.concatenate(
            [jnp.broadcast_to(ex0, (b, dh)), jnp.broadcast_to(ex1, (b, dh))], axis=1
        )
        o_ref[...] = jnp.concatenate(
            [v_mat * scale, ex0, ex1, jnp.zeros((b, dext - 2 * dh - 2), F32)], axis=1
        )

    wspec = lambda shape: pl.BlockSpec(shape, lambda i: (0, 0))
    return pl.pallas_call(
        body,
        grid=(e // b,),
        in_specs=[
            pl.BlockSpec((b, 128), lambda i: (i + row_off, 0)),
            pl.BlockSpec((b, 128), lambda i: (i, 0)),
            pl.BlockSpec((b, de), lambda i: (i, 0)),
            pl.BlockSpec((b, 1), lambda i: (i, 0)),
            wspec((128, 128)), wspec((de, 128)), wspec((128, 128)), wspec((1, 128)),
            wspec((128, 128)), wspec((de, 128)), wspec((128, 128)), wspec((1, 128)),
            wspec((1, 128)), wspec((1, 128)),
        ],
        out_specs=pl.BlockSpec((b, dext), lambda i: (i, 0)),
        out_shape=jax.ShapeDtypeStruct((e, dext), F32),
        compiler_params=pltpu.CompilerParams(
            dimension_semantics=("parallel",)
        ),
    )(src_data, qe, edge_feat, dt2,
      wkn_t, wke_t, wkt_t, bk2, wvn_t, wve_t, wvt_t, bv2, twp, tbp)


def _tc_final(p0, p1, src_data, wod_t, wos_t, bo2, g2, b2, nd, b, dh, dext):
    def body(p0_ref, p1_ref, q_ref, wod, wos, bor, gr, br, o_ref):
        s = p0_ref[...] + p1_ref[...]
        num = s[:, : 2 * dh]
        den0 = jnp.maximum(s[:, 2 * dh : 2 * dh + 1], 1e-16)
        den1 = jnp.maximum(s[:, 2 * dh + 1 : 2 * dh + 2], 1e-16)
        den = jnp.concatenate(
            [jnp.broadcast_to(den0, (b, dh)), jnp.broadcast_to(den1, (b, dh))], axis=1
        )
        dst_h = num / den
        r = (
            jnp.dot(dst_h, wod[...], preferred_element_type=F32)
            + jnp.dot(q_ref[...], wos[...], preferred_element_type=F32)
            + bor[...]
        )
        r = jnp.maximum(r, 0.0)
        mu = jnp.mean(r, axis=1, keepdims=True)
        var = jnp.mean((r - mu) ** 2, axis=1, keepdims=True)
        o_ref[...] = (r - mu) / jnp.sqrt(var + 1e-5) * gr[...] + br[...]

    wspec = lambda shape: pl.BlockSpec(shape, lambda i: (0, 0))
    return pl.pallas_call(
        body,
        grid=(nd // b,),
        in_specs=[
            pl.BlockSpec((b, dext), lambda i: (i, 0)),
            pl.BlockSpec((b, dext), lambda i: (i, 0)),
            pl.BlockSpec((b, 128), lambda i: (i, 0)),
            wspec((128, 128)), wspec((128, 128)),
            wspec((1, 128)), wspec((1, 128)), wspec((1, 128)),
        ],
        out_specs=pl.BlockSpec((b, 128), lambda i: (i, 0)),
        out_shape=jax.ShapeDtypeStruct((nd, 128), F32),
        compiler_params=pltpu.CompilerParams(
            dimension_semantics=("parallel",)
        ),
    )(p0, p1, src_data, wod_t, wos_t, bo2, g2, b2)


def kernel(h, src_idx, edge_dt, edge_feat, edge_dst, num_dst, time_w, time_b,
           Wq, bq, Wk, bk, Wv, bv, Wo, bo, ln_g, ln_b):
    e = edge_dst.shape[0]
    n_src, dn = h.shape
    nd = n_src - e
    dt_dim = time_w.shape[0]
    dout = Wq.shape[0]
    dh = dout // 2
    de = edge_feat.shape[1]
    dext = 144
    b = 400
    k = 80
    nw = 32

    wqn_t = Wq[:, :dn].T
    bqp = (bq + jnp.cos(time_b) @ Wq[:, dn:].T).reshape(1, dout)
    wkn_t = Wk[:, :dn].T
    wke_t = Wk[:, dn : dn + de].T
    wkt_t = jnp.pad(Wk[:, dn + de :].T, ((0, 128 - dt_dim), (0, 0)))
    wvn_t = Wv[:, :dn].T
    wve_t = Wv[:, dn : dn + de].T
    wvt_t = jnp.pad(Wv[:, dn + de :].T, ((0, 128 - dt_dim), (0, 0)))
    twp = jnp.pad(time_w[:, 0], (0, 128 - dt_dim)).reshape(1, 128)
    tbp = jnp.pad(time_b, (0, 128 - dt_dim)).reshape(1, 128)
    wod_t = Wo[:, :dout].T
    wos_t = Wo[:, dout:].T
    bo2 = bo.reshape(1, dout)
    g2 = ln_g.reshape(1, dout)
    b2 = ln_b.reshape(1, dout)
    bk2 = bk.reshape(1, dout)
    bv2 = bv.reshape(1, dout)

    chunk = nw * k
    n_pad = ((n_src + chunk - 1) // chunk) * chunk
    si = jnp.pad(src_idx.astype(I32), (0, n_pad - n_src))
    src_data = _sc_gather(h, si, k)

    qn = _tc_qn(src_data, wqn_t, bqp, nd, b)

    dst_i = edge_dst.astype(I32)
    qe = _sc_gather(qn, dst_i, k)

    dt2 = edge_dt.reshape(e, 1)
    ext = _tc_edge_pass(src_data, qe, edge_feat, dt2,
                        (wkn_t, wke_t, wkt_t, bk2), (wvn_t, wve_t, wvt_t, bv2),
                        twp, tbp, e, nd, b, dh, dext)

    p0, p1 = _sc_scatter_add(ext, dst_i, nd, k)

    return _tc_final(p0, p1, src_data, wod_t, wos_t, bo2, g2, b2, nd, b, dh, dext)


# speedup vs baseline: 2.8330x; 2.8330x over previous
import functools

import jax
import jax.numpy as jnp
from jax import lax
from jax.experimental import pallas as pl
from jax.experimental.pallas import tpu as pltpu
from jax.experimental.pallas import tpu_sc as plsc

F32 = jnp.float32
I32 = jnp.int32


def _sc_gather(table, idx, k):
    n = idx.shape[0]
    d = table.shape[1]
    info = plsc.get_sparse_core_info()
    nc, ns = info.num_cores, info.num_subcores
    nw = nc * ns
    per_w = n // nw
    nblk = per_w // k

    mesh = plsc.VectorSubcoreMesh(core_axis_name="c", subcore_axis_name="s")

    @functools.partial(
        pl.kernel,
        out_type=jax.ShapeDtypeStruct((n, d), F32),
        mesh=mesh,
        scratch_types=[
            pltpu.VMEM((k,), I32),
            pltpu.VMEM((k, d), F32),
            pltpu.SemaphoreType.DMA,
        ],
    )
    def gk(table_hbm, idx_hbm, out_hbm, idx_v, rows_v, sem):
        wid = lax.axis_index("s") * nc + lax.axis_index("c")
        base = wid * per_w

        def body(j, carry):
            off = base + j * k
            pltpu.sync_copy(idx_hbm.at[pl.ds(off, k)], idx_v)
            pltpu.async_copy(table_hbm.at[idx_v], rows_v, sem).wait()
            pltpu.sync_copy(rows_v, out_hbm.at[pl.ds(off, k)])
            return carry

        lax.fori_loop(0, nblk, body, 0)

    return gk(table, idx)


def _sc_scatter_add(ext0, ext1, dst_idx, nd, k):
    e, dext = ext0.shape
    info = plsc.get_sparse_core_info()
    nc, ns = info.num_cores, info.num_subcores
    per_tile = e // ns
    nblk = per_tile // k
    rows_per_tile = ((nd + 8 * ns - 1) // (8 * ns)) * 8
    nd_pad = rows_per_tile * ns

    zeros = jnp.zeros((rows_per_tile, dext), F32)
    mesh = plsc.VectorSubcoreMesh(core_axis_name="c", subcore_axis_name="s")

    @functools.partial(
        pl.kernel,
        out_type=(
            jax.ShapeDtypeStruct((nd_pad, dext), F32),
            jax.ShapeDtypeStruct((nd_pad, dext), F32),
        ),
        mesh=mesh,
        scratch_types=[
            pltpu.VMEM((k,), I32),
            pltpu.VMEM((k, dext), F32),
            pltpu.SemaphoreType.DMA,
            pltpu.VMEM_SHARED((nd_pad, dext), F32),
        ],
    )
    def sk(e0_hbm, e1_hbm, dst_hbm, z_hbm, out0, out1, idx_v, rows_v, sem, acc):
        cid = lax.axis_index("c")
        sid = lax.axis_index("s")
        my_rows = pl.ds(sid * rows_per_tile, rows_per_tile)
        pltpu.sync_copy(z_hbm, acc.at[my_rows])
        plsc.subcore_barrier()

        base = sid * per_tile

        def body(ext_hbm):
            def step(j, carry):
                off = base + j * k
                pltpu.sync_copy(dst_hbm.at[pl.ds(off, k)], idx_v)
                pltpu.sync_copy(ext_hbm.at[pl.ds(off, k)], rows_v)
                pltpu.sync_copy(rows_v, acc.at[idx_v], add=True)
                return carry

            lax.fori_loop(0, nblk, step, 0)

        @pl.when(cid == 0)
        def _():
            body(e0_hbm)

        @pl.when(cid == 1)
        def _():
            body(e1_hbm)

        plsc.subcore_barrier()

        @pl.when(cid == 0)
        def _():
            pltpu.sync_copy(acc.at[my_rows], out0.at[my_rows])

        @pl.when(cid == 1)
        def _():
            pltpu.sync_copy(acc.at[my_rows], out1.at[my_rows])

    return sk(ext0, ext1, dst_idx, zeros)


def _tc_qn(src_data, wqn_t, bqp, nd, b):
    def body(q_ref, w_ref, b_ref, o_ref):
        o_ref[...] = (
            jnp.dot(q_ref[...], w_ref[...], preferred_element_type=F32) + b_ref[...]
        )

    return pl.pallas_call(
        body,
        grid=(nd // b,),
        in_specs=[
            pl.BlockSpec((b, 128), lambda i: (i, 0)),
            pl.BlockSpec((128, 128), lambda i: (0, 0)),
            pl.BlockSpec((1, 128), lambda i: (0, 0)),
        ],
        out_specs=pl.BlockSpec((b, 128), lambda i: (i, 0)),
        out_shape=jax.ShapeDtypeStruct((nd, 128), F32),
        compiler_params=pltpu.CompilerParams(
            dimension_semantics=("parallel",)
        ),
    )(src_data, wqn_t, bqp)


def _tc_edge_pass(src_data, qe, edge_feat, dt2, wk_parts, wv_parts, twp, tbp,
                  e, nd, b, dh, dext):
    wkn_t, wke_t, wkt_t, bk2 = wk_parts
    wvn_t, wve_t, wvt_t, bv2 = wv_parts
    row_off = nd // b
    de = edge_feat.shape[1]

    def body(kv_ref, qe_ref, ef_ref, dt_ref,
             wkn, wke, wkt, bkr, wvn, wve, wvt, bvr, twr, tbr, o0_ref, o1_ref):
        tf = jnp.cos(dt_ref[...] * twr[...] + tbr[...])
        kv = kv_ref[...]
        ef = ef_ref[...]
        k_mat = (
            jnp.dot(kv, wkn[...], preferred_element_type=F32)
            + jnp.dot(ef, wke[...], preferred_element_type=F32)
            + jnp.dot(tf, wkt[...], preferred_element_type=F32)
            + bkr[...]
        )
        v_mat = (
            jnp.dot(kv, wvn[...], preferred_element_type=F32)
            + jnp.dot(ef, wve[...], preferred_element_type=F32)
            + jnp.dot(tf, wvt[...], preferred_element_type=F32)
            + bvr[...]
        )
        s = qe_ref[...] * k_mat
        d0 = jnp.sum(s[:, :dh], axis=1, keepdims=True)
        d1 = jnp.sum(s[:, dh:], axis=1, keepdims=True)
        l0 = jnp.where(d0 >= 0, d0, 0.2 * d0)
        l1 = jnp.where(d1 >= 0, d1, 0.2 * d1)
        ex0 = jnp.exp(jnp.minimum(l0, 75.0))
        ex1 = jnp.exp(jnp.minimum(l1, 75.0))
        z = jnp.zeros((b, dext - dh - 1), F32)
        o0_ref[...] = jnp.concatenate(
            [v_mat[:, :dh] * jnp.broadcast_to(ex0, (b, dh)), ex0, z], axis=1
        )
        o1_ref[...] = jnp.concatenate(
            [v_mat[:, dh:] * jnp.broadcast_to(ex1, (b, dh)), ex1, z], axis=1
        )

    wspec = lambda shape: pl.BlockSpec(shape, lambda i: (0, 0))
    return pl.pallas_call(
        body,
        grid=(e // b,),
        in_specs=[
            pl.BlockSpec((b, 128), lambda i: (i + row_off, 0)),
            pl.BlockSpec((b, 128), lambda i: (i, 0)),
            pl.BlockSpec((b, de), lambda i: (i, 0)),
            pl.BlockSpec((b, 1), lambda i: (i, 0)),
            wspec((128, 128)), wspec((de, 128)), wspec((128, 128)), wspec((1, 128)),
            wspec((128, 128)), wspec((de, 128)), wspec((128, 128)), wspec((1, 128)),
            wspec((1, 128)), wspec((1, 128)),
        ],
        out_specs=(
            pl.BlockSpec((b, dext), lambda i: (i, 0)),
            pl.BlockSpec((b, dext), lambda i: (i, 0)),
        ),
        out_shape=(
            jax.ShapeDtypeStruct((e, dext), F32),
            jax.ShapeDtypeStruct((e, dext), F32),
        ),
        compiler_params=pltpu.CompilerParams(
            dimension_semantics=("parallel",)
        ),
    )(src_data, qe, edge_feat, dt2,
      wkn_t, wke_t, wkt_t, bk2, wvn_t, wve_t, wvt_t, bv2, twp, tbp)


def _tc_final(p0, p1, src_data, wod_t, wos_t, bo2, g2, b2, nd, b, dh, dext):
    def body(p0_ref, p1_ref, q_ref, wod, wos, bor, gr, br, o_ref):
        p0 = p0_ref[...]
        p1 = p1_ref[...]
        den0 = jnp.maximum(p0[:, dh : dh + 1], 1e-16)
        den1 = jnp.maximum(p1[:, dh : dh + 1], 1e-16)
        dst_h = jnp.concatenate(
            [p0[:, :dh] / jnp.broadcast_to(den0, (b, dh)),
             p1[:, :dh] / jnp.broadcast_to(den1, (b, dh))], axis=1
        )
        r = (
            jnp.dot(dst_h, wod[...], preferred_element_type=F32)
            + jnp.dot(q_ref[...], wos[...], preferred_element_type=F32)
            + bor[...]
        )
        r = jnp.maximum(r, 0.0)
        mu = jnp.mean(r, axis=1, keepdims=True)
        var = jnp.mean((r - mu) ** 2, axis=1, keepdims=True)
        o_ref[...] = (r - mu) / jnp.sqrt(var + 1e-5) * gr[...] + br[...]

    wspec = lambda shape: pl.BlockSpec(shape, lambda i: (0, 0))
    return pl.pallas_call(
        body,
        grid=(nd // b,),
        in_specs=[
            pl.BlockSpec((b, dext), lambda i: (i, 0)),
            pl.BlockSpec((b, dext), lambda i: (i, 0)),
            pl.BlockSpec((b, 128), lambda i: (i, 0)),
            wspec((128, 128)), wspec((128, 128)),
            wspec((1, 128)), wspec((1, 128)), wspec((1, 128)),
        ],
        out_specs=pl.BlockSpec((b, 128), lambda i: (i, 0)),
        out_shape=jax.ShapeDtypeStruct((nd, 128), F32),
        compiler_params=pltpu.CompilerParams(
            dimension_semantics=("parallel",)
        ),
    )(p0, p1, src_data, wod_t, wos_t, bo2, g2, b2)


def kernel(h, src_idx, edge_dt, edge_feat, edge_dst, num_dst, time_w, time_b,
           Wq, bq, Wk, bk, Wv, bv, Wo, bo, ln_g, ln_b):
    e = edge_dst.shape[0]
    n_src, dn = h.shape
    nd = n_src - e
    dt_dim = time_w.shape[0]
    dout = Wq.shape[0]
    dh = dout // 2
    de = edge_feat.shape[1]
    dext = 128
    b = 400
    k = 80
    nw = 32

    wqn_t = Wq[:, :dn].T
    bqp = (bq + jnp.cos(time_b) @ Wq[:, dn:].T).reshape(1, dout)
    wkn_t = Wk[:, :dn].T
    wke_t = Wk[:, dn : dn + de].T
    wkt_t = jnp.pad(Wk[:, dn + de :].T, ((0, 128 - dt_dim), (0, 0)))
    wvn_t = Wv[:, :dn].T
    wve_t = Wv[:, dn : dn + de].T
    wvt_t = jnp.pad(Wv[:, dn + de :].T, ((0, 128 - dt_dim), (0, 0)))
    twp = jnp.pad(time_w[:, 0], (0, 128 - dt_dim)).reshape(1, 128)
    tbp = jnp.pad(time_b, (0, 128 - dt_dim)).reshape(1, 128)
    wod_t = Wo[:, :dout].T
    wos_t = Wo[:, dout:].T
    bo2 = bo.reshape(1, dout)
    g2 = ln_g.reshape(1, dout)
    b2 = ln_b.reshape(1, dout)
    bk2 = bk.reshape(1, dout)
    bv2 = bv.reshape(1, dout)

    chunk = nw * k
    n_pad = ((n_src + chunk - 1) // chunk) * chunk
    si = jnp.pad(src_idx.astype(I32), (0, n_pad - n_src))
    src_data = _sc_gather(h, si, k)

    qn = _tc_qn(src_data, wqn_t, bqp, nd, b)

    dst_i = edge_dst.astype(I32)
    qe = _sc_gather(qn, dst_i, k)

    dt2 = edge_dt.reshape(e, 1)
    ext0, ext1 = _tc_edge_pass(src_data, qe, edge_feat, dt2,
                               (wkn_t, wke_t, wkt_t, bk2), (wvn_t, wve_t, wvt_t, bv2),
                               twp, tbp, e, nd, b, dh, dext)

    p0, p1 = _sc_scatter_add(ext0, ext1, dst_i, nd, k)

    return _tc_final(p0, p1, src_data, wod_t, wos_t, bo2, g2, b2, nd, b, dh, dext)
